# baseline (speedup 1.0000x reference)
import functools
import math

import jax
import jax.numpy as jnp
from jax import lax
from jax.experimental import pallas as pl
from jax.experimental.pallas import tpu as pltpu
from jax.experimental.pallas import tpu_sc as plsc

_SCALE = math.sqrt(32.0)
_D1, _D2, _D3 = 4, 10, 20
_BB, _BL = 8, 128
_NROWS = _D2 * _D3 * _BB * _D1
_BLK = 8


@jax.jit
def _sc_impl(xp, table):
    info = plsc.get_sparse_core_info()
    num_cores, num_subcores = info.num_cores, info.num_subcores
    num_workers = num_cores * num_subcores
    rows_per_w = _NROWS // num_workers
    n_blocks = rows_per_w // _BLK

    mesh = plsc.VectorSubcoreMesh(core_axis_name="c", subcore_axis_name="s")

    @functools.partial(
        pl.kernel,
        out_type=jax.ShapeDtypeStruct((_D1, _D2, _D3, 8, _BB, 8, _BL),
                                      jnp.float32),
        mesh=mesh,
        scratch_types=[
            pltpu.VMEM((_BLK, _BL), jnp.int32),
            pltpu.VMEM((_BLK * _BL, 32), jnp.float32),
            pltpu.VMEM((_BLK, 4, 8, _BL), jnp.float32),
            pltpu.SemaphoreType.DMA,
            pltpu.SemaphoreType.DMA,
        ],
        compiler_params=pltpu.CompilerParams(use_tc_tiling_on_sc=False,
                                             needs_layout_passes=False),
    )
    def k(xp_hbm, tab_hbm, out_hbm, idx_v, rows_v, tiles_v, gsem, wsem):
        wid = lax.axis_index("s") * num_cores + lax.axis_index("c")
        w_row0 = wid * rows_per_w
        lane = lax.broadcasted_iota(jnp.int32, (16,), 0)

        def do_block(i, carry):
            r0 = w_row0 + i * _BLK
            pltpu.sync_copy(xp_hbm.at[pl.ds(r0, _BLK)], idx_v)
            gathers = []
            for j in range(_BLK):
                gathers.append(pltpu.async_copy(
                    tab_hbm.at[idx_v.at[j]],
                    rows_v.at[pl.ds(j * _BL, _BL)], gsem))
            @pl.when(i > 0)
            def _():
                for su in range(_BLK):
                    for h in range(2):
                        pltpu.make_async_copy(
                            tiles_v.at[su],
                            out_hbm.at[0, 0, 0, pl.ds(4 * h, 4), 0],
                            wsem).wait()
            for g in gathers:
                g.wait()

            def do_su(su, c):
                for blk in range(8):
                    row_ids = su * _BL + blk * 16 + lane

                    def do_el(el, c2):
                        for ehp in range(4):
                            col = ehp * 8 + el
                            col_ids = jnp.full((16,), col, jnp.int32)
                            v = plsc.load_gather(rows_v, [row_ids, col_ids])
                            tiles_v[su, ehp, el, pl.ds(blk * 16, 16)] = (
                                v * _SCALE)
                        return c2

                    lax.fori_loop(0, 8, do_el, 0)
                return c

            lax.fori_loop(0, _BLK, do_su, 0)

            def do_wr(su, c):
                r = r0 + su
                d1 = r % _D1
                bb = (r // _D1) % _BB
                t = r // (_D1 * _BB)
                d3 = t % _D3
                d2 = t // _D3
                for h in range(2):
                    pltpu.async_copy(
                        tiles_v.at[su],
                        out_hbm.at[d1, d2, d3, pl.ds(4 * h, 4), bb], wsem)
                return c

            lax.fori_loop(0, _BLK, do_wr, 0)
            return carry

        lax.fori_loop(0, n_blocks, do_block, 0)
        for su in range(_BLK):
            for h in range(2):
                pltpu.make_async_copy(
                    tiles_v.at[su],
                    out_hbm.at[0, 0, 0, pl.ds(4 * h, 4), 0], wsem).wait()

    return k(xp, table)


def kernel(x, embedding):
    xp = (x.reshape(_BB, _BL, _D1, _D2, _D3)
            .transpose(3, 4, 0, 2, 1)
            .reshape(_NROWS, _BL)
            .astype(jnp.int32))
    out7 = _sc_impl(xp, embedding)
    out = (out7.transpose(4, 6, 0, 1, 2, 3, 5)
               .reshape(_BB * _BL, _D1, _D2, _D3, 64))
    return out


# speedup vs baseline: 1.0331x; 1.0331x over previous
import functools
import math

import jax
import jax.numpy as jnp
from jax import lax
from jax.experimental import pallas as pl
from jax.experimental.pallas import tpu as pltpu
from jax.experimental.pallas import tpu_sc as plsc

_SCALE = math.sqrt(32.0)
_D1, _D2, _D3 = 4, 10, 20
_BB, _BL = 8, 128
_NROWS = _D2 * _D3 * _BB * _D1
_BLK = 4


@jax.jit
def _sc_impl(xp, table):
    info = plsc.get_sparse_core_info()
    num_cores, num_subcores = info.num_cores, info.num_subcores
    num_workers = num_cores * num_subcores
    rows_per_w = _NROWS // num_workers
    n_blocks = rows_per_w // _BLK

    mesh = plsc.VectorSubcoreMesh(core_axis_name="c", subcore_axis_name="s")

    @functools.partial(
        pl.kernel,
        out_type=jax.ShapeDtypeStruct((_D1, _D2, _D3, 8, _BB, 8, _BL),
                                      jnp.float32),
        mesh=mesh,
        scratch_types=[
            pltpu.VMEM((2, _BLK, _BL), jnp.int32),
            pltpu.VMEM((2, _BLK * _BL, 32), jnp.float32),
            pltpu.VMEM((2, _BLK, 4, 8, _BL), jnp.float32),
            pltpu.SemaphoreType.DMA,
            pltpu.SemaphoreType.DMA,
            pltpu.SemaphoreType.DMA,
            pltpu.SemaphoreType.DMA,
        ],
        compiler_params=pltpu.CompilerParams(use_tc_tiling_on_sc=False,
                                             needs_layout_passes=False),
    )
    def k(xp_hbm, tab_hbm, out_hbm, idx_v, rows_v, tiles_v,
          gsem0, gsem1, wsem0, wsem1):
        gsems = (gsem0, gsem1)
        wsems = (wsem0, wsem1)
        wid = lax.axis_index("s") * num_cores + lax.axis_index("c")
        w_row0 = wid * rows_per_w
        lane = lax.broadcasted_iota(jnp.int32, (16,), 0)

        def load_and_fire(u, p):
            r0 = w_row0 + u * _BLK
            pltpu.sync_copy(xp_hbm.at[pl.ds(r0, _BLK)], idx_v.at[p])
            for j in range(_BLK):
                pltpu.async_copy(
                    tab_hbm.at[idx_v.at[p, j]],
                    rows_v.at[p, pl.ds(j * _BL, _BL)], gsems[p])

        def wait_gathers(p):
            for j in range(_BLK):
                pltpu.make_async_copy(
                    tab_hbm.at[idx_v.at[p, j]],
                    rows_v.at[p, pl.ds(j * _BL, _BL)], gsems[p]).wait()

        def fire_writes(u, p):
            for su in range(_BLK):
                r = w_row0 + u * _BLK + su
                d1 = r % _D1
                bb = (r // _D1) % _BB
                t = r // (_D1 * _BB)
                d3 = t % _D3
                d2 = t // _D3
                for h in range(2):
                    pltpu.async_copy(
                        tiles_v.at[p, su],
                        out_hbm.at[d1, d2, d3, pl.ds(4 * h, 4), bb],
                        wsems[p])

        def drain_writes(p):
            for su in range(_BLK):
                for h in range(2):
                    pltpu.make_async_copy(
                        tiles_v.at[p, su],
                        out_hbm.at[0, 0, 0, pl.ds(4 * h, 4), 0],
                        wsems[p]).wait()

        def transpose(p):
            for su in range(_BLK):
                def do_grp(blk, c):
                    row_ids = su * _BL + blk * 16 + lane
                    for ehp in range(4):
                        for el in range(8):
                            col = ehp * 8 + el
                            col_ids = jnp.full((16,), col, jnp.int32)
                            v = plsc.load_gather(rows_v.at[p],
                                                 [row_ids, col_ids])
                            tiles_v[p, su, ehp, el, pl.ds(blk * 16, 16)] = (
                                v * _SCALE)
                    return c
                lax.fori_loop(0, 8, do_grp, 0)

        load_and_fire(0, 0)
        load_and_fire(1, 1)

        def do_pair(g, carry):
            for p in range(2):
                u = 2 * g + p
                wait_gathers(p)

                @pl.when(u >= 2)
                def _():
                    drain_writes(p)

                transpose(p)
                fire_writes(u, p)

                @pl.when(u + 2 < n_blocks)
                def _():
                    load_and_fire(u + 2, p)
            return carry

        lax.fori_loop(0, n_blocks // 2, do_pair, 0)
        drain_writes(0)
        drain_writes(1)

    return k(xp, table)


def kernel(x, embedding):
    xp = (x.reshape(_BB, _BL, _D1, _D2, _D3)
            .transpose(3, 4, 0, 2, 1)
            .reshape(_NROWS, _BL)
            .astype(jnp.int32))
    out7 = _sc_impl(xp, embedding)
    out = (out7.transpose(4, 6, 0, 1, 2, 3, 5)
               .reshape(_BB * _BL, _D1, _D2, _D3, 64))
    return out


# speedup vs baseline: 1.6973x; 1.6429x over previous
import functools
import math

import jax
import jax.numpy as jnp
from jax import lax
from jax.experimental import pallas as pl
from jax.experimental.pallas import tpu as pltpu
from jax.experimental.pallas import tpu_sc as plsc

_SCALE = math.sqrt(32.0)
_D1, _D2, _D3 = 4, 10, 20
_BB, _BL = 8, 128
_NROWS = _D2 * _D3 * _BB * _D1
_BLK = 4


@jax.jit
def _sc_impl(xp, table):
    info = plsc.get_sparse_core_info()
    num_cores, num_subcores = info.num_cores, info.num_subcores
    num_workers = num_cores * num_subcores
    rows_per_w = _NROWS // num_workers
    n_blocks = rows_per_w // _BLK

    mesh = plsc.VectorSubcoreMesh(core_axis_name="c", subcore_axis_name="s")

    @functools.partial(
        pl.kernel,
        out_type=jax.ShapeDtypeStruct((_D1, _D2, _D3, 8, _BB, 8, _BL),
                                      jnp.float32),
        mesh=mesh,
        scratch_types=[
            pltpu.VMEM((2, _BLK, _BL), jnp.int32),
            pltpu.VMEM((2, _BLK * _BL, 32), jnp.float32),
            pltpu.VMEM((2, _BLK, 4, 8, 129), jnp.float32),
            pltpu.SemaphoreType.DMA,
            pltpu.SemaphoreType.DMA,
            pltpu.SemaphoreType.DMA,
            pltpu.SemaphoreType.DMA,
        ],
        compiler_params=pltpu.CompilerParams(use_tc_tiling_on_sc=False,
                                             needs_layout_passes=False),
    )
    def k(xp_hbm, tab_hbm, out_hbm, idx_v, rows_v, tiles_v,
          gsem0, gsem1, wsem0, wsem1):
        gsems = (gsem0, gsem1)
        wsems = (wsem0, wsem1)
        wid = lax.axis_index("s") * num_cores + lax.axis_index("c")
        w_row0 = wid * rows_per_w
        lane = lax.broadcasted_iota(jnp.int32, (16,), 0)

        def load_and_fire(u, p):
            r0 = w_row0 + u * _BLK
            pltpu.sync_copy(xp_hbm.at[pl.ds(r0, _BLK)], idx_v.at[p])
            for j in range(_BLK):
                pltpu.async_copy(
                    tab_hbm.at[idx_v.at[p, j]],
                    rows_v.at[p, pl.ds(j * _BL, _BL)], gsems[p])

        def wait_gathers(p):
            for j in range(_BLK):
                pltpu.make_async_copy(
                    tab_hbm.at[idx_v.at[p, j]],
                    rows_v.at[p, pl.ds(j * _BL, _BL)], gsems[p]).wait()

        def fire_writes(u, p):
            for su in range(_BLK):
                r = w_row0 + u * _BLK + su
                d1 = r % _D1
                bb = (r // _D1) % _BB
                t = r // (_D1 * _BB)
                d3 = t % _D3
                d2 = t // _D3
                for h in range(2):
                    pltpu.async_copy(
                        tiles_v.at[p, su, :, :, pl.ds(0, _BL)],
                        out_hbm.at[d1, d2, d3, pl.ds(4 * h, 4), bb],
                        wsems[p])

        def drain_writes(p):
            for su in range(_BLK):
                for h in range(2):
                    pltpu.make_async_copy(
                        tiles_v.at[p, su, :, :, pl.ds(0, _BL)],
                        out_hbm.at[0, 0, 0, pl.ds(4 * h, 4), 0],
                        wsems[p]).wait()

        ehp_lo, el_lo = lane // 8, lane % 8
        ehp_hi = ehp_lo + 2

        def transpose(p):
            for su in range(_BLK):
                def do_row(b, c):
                    for s in range(4):
                        bq = b * 4 + s
                        row = su * _BL + bq
                        bl_ids = jnp.full((16,), bq, jnp.int32)
                        v0 = rows_v[p, row, pl.ds(0, 16)] * _SCALE
                        v1 = rows_v[p, row, pl.ds(16, 16)] * _SCALE
                        plsc.store_scatter(tiles_v.at[p, su],
                                           [ehp_lo, el_lo, bl_ids], v0)
                        plsc.store_scatter(tiles_v.at[p, su],
                                           [ehp_hi, el_lo, bl_ids], v1)
                    return c
                lax.fori_loop(0, _BL // 4, do_row, 0)

        load_and_fire(0, 0)
        load_and_fire(1, 1)

        def do_pair(g, carry):
            for p in range(2):
                u = 2 * g + p
                wait_gathers(p)

                @pl.when(u >= 2)
                def _():
                    drain_writes(p)

                transpose(p)
                fire_writes(u, p)

                @pl.when(u + 2 < n_blocks)
                def _():
                    load_and_fire(u + 2, p)
            return carry

        lax.fori_loop(0, n_blocks // 2, do_pair, 0)
        drain_writes(0)
        drain_writes(1)

    return k(xp, table)


def kernel(x, embedding):
    xp = (x.reshape(_BB, _BL, _D1, _D2, _D3)
            .transpose(3, 4, 0, 2, 1)
            .reshape(_NROWS, _BL)
            .astype(jnp.int32))
    out7 = _sc_impl(xp, embedding)
    out = (out7.transpose(4, 6, 0, 1, 2, 3, 5)
               .reshape(_BB * _BL, _D1, _D2, _D3, 64))
    return out


# speedup vs baseline: 2.2555x; 1.3289x over previous
import functools
import math

import jax
import jax.numpy as jnp
from jax import lax
from jax.experimental import pallas as pl
from jax.experimental.pallas import tpu as pltpu
from jax.experimental.pallas import tpu_sc as plsc

_SCALE = math.sqrt(32.0)
_D1, _D2, _D3 = 4, 10, 20
_BB, _BL = 8, 128
_NROWS = _D2 * _D3 * _BB * _D1
_BLK = 4

_E = 1_000_000
_NFULL = _E // 128
_REM = _E - _NFULL * 128


@jax.jit
def _sc_table_transpose(tab_t, tail16):
    info = plsc.get_sparse_core_info()
    num_workers = info.num_cores * info.num_subcores
    n_k = (_NFULL + num_workers - 1) // num_workers
    n_pairs = (n_k + 1) // 2

    mesh = plsc.VectorSubcoreMesh(core_axis_name="c", subcore_axis_name="s")

    @functools.partial(
        pl.kernel,
        out_type=jax.ShapeDtypeStruct((_E // 4, 128), jnp.float32),
        mesh=mesh,
        scratch_types=[
            pltpu.VMEM((2, 32, _BL), jnp.float32),
            pltpu.VMEM((2, 32, _BL), jnp.float32),
            pltpu.VMEM((16, _BL), jnp.float32),
            pltpu.SemaphoreType.DMA,
            pltpu.SemaphoreType.DMA,
            pltpu.SemaphoreType.DMA,
            pltpu.SemaphoreType.DMA,
        ],
        compiler_params=pltpu.CompilerParams(use_tc_tiling_on_sc=True,
                                             needs_layout_passes=False),
    )
    def tk(tab_hbm, tail_hbm, out_hbm, in_v, out_v, tail_v,
           isem0, isem1, wsem0, wsem1):
        isems = (isem0, isem1)
        wsems = (wsem0, wsem1)
        wid = lax.axis_index("s") * info.num_cores + lax.axis_index("c")
        lane = lax.broadcasted_iota(jnp.int32, (16,), 0)

        def group_of(k):
            return k * num_workers + wid

        def in_src(g):
            c0 = pl.multiple_of(g * _BL, _BL)
            return tab_hbm.at[:, pl.ds(c0, _BL)]

        def load(k, p):
            g = group_of(k)

            @pl.when(g < _NFULL)
            def _():
                pltpu.async_copy(in_src(g), in_v.at[p], isems[p])

        def wait_in(g, p):
            pltpu.make_async_copy(in_src(g), in_v.at[p], isems[p]).wait()

        def transpose(src, dst, ncols):
            for e0 in range(0, 32, 16):
                e_ids = e0 + lane

                def do_blk(t, c):
                    i0 = t * 16

                    def do_d(d, c2):
                        i_ids = i0 + ((lane + d) & 15)
                        v = plsc.load_gather(src, [e_ids, i_ids])
                        f = i_ids * 32 + e_ids
                        plsc.store_scatter(dst, [f >> 7, f & 127], v)
                        return c2
                    lax.fori_loop(0, 16, do_d, c)
                    return c
                lax.fori_loop(0, ncols // 16, do_blk, 0)

        def fire_write(g, p):
            r0 = pl.multiple_of(g * 32, 32)
            pltpu.async_copy(out_v.at[p],
                             out_hbm.at[pl.ds(r0, 32)], wsems[p])

        def drain_write(p):
            pltpu.make_async_copy(out_v.at[p],
                                  out_hbm.at[pl.ds(0, 32)], wsems[p]).wait()

        load(0, 0)
        load(1, 1)

        def do_pair(j, carry):
            for p in range(2):
                k = 2 * j + p
                g = group_of(k)

                @pl.when(g < _NFULL)
                def _():
                    wait_in(g, p)

                    @pl.when(k >= 2)
                    def _():
                        drain_write(p)

                    transpose(in_v.at[p], out_v.at[p], _BL)
                    fire_write(g, p)
                load(k + 2, p)
            return carry

        lax.fori_loop(0, n_pairs, do_pair, 0)

        drain_write(0)
        drain_write(1)

        @pl.when(wid == 0)
        def _():
            pltpu.sync_copy(tail_hbm, tail_v)
            pltpu.sync_copy(tail_v, out_hbm.at[pl.ds(_NFULL * 32, 16)])

    return tk(tab_t, tail16)


@jax.jit
def _sc_impl(xp, table):
    info = plsc.get_sparse_core_info()
    num_cores, num_subcores = info.num_cores, info.num_subcores
    num_workers = num_cores * num_subcores
    rows_per_w = _NROWS // num_workers
    n_blocks = rows_per_w // _BLK

    mesh = plsc.VectorSubcoreMesh(core_axis_name="c", subcore_axis_name="s")

    @functools.partial(
        pl.kernel,
        out_type=jax.ShapeDtypeStruct((_D1, _D2, _D3, 8, _BB, 8, _BL),
                                      jnp.float32),
        mesh=mesh,
        scratch_types=[
            pltpu.VMEM((2, _BLK, _BL), jnp.int32),
            pltpu.VMEM((2, _BLK * _BL, 32), jnp.float32),
            pltpu.VMEM((2, _BLK, 4, 8, 129), jnp.float32),
            pltpu.SemaphoreType.DMA,
            pltpu.SemaphoreType.DMA,
            pltpu.SemaphoreType.DMA,
            pltpu.SemaphoreType.DMA,
        ],
        compiler_params=pltpu.CompilerParams(use_tc_tiling_on_sc=False,
                                             needs_layout_passes=False),
    )
    def k(xp_hbm, tab_hbm, out_hbm, idx_v, rows_v, tiles_v,
          gsem0, gsem1, wsem0, wsem1):
        gsems = (gsem0, gsem1)
        wsems = (wsem0, wsem1)
        wid = lax.axis_index("s") * num_cores + lax.axis_index("c")
        w_row0 = wid * rows_per_w
        lane = lax.broadcasted_iota(jnp.int32, (16,), 0)

        def load_and_fire(u, p):
            r0 = w_row0 + u * _BLK
            pltpu.sync_copy(xp_hbm.at[pl.ds(r0, _BLK)], idx_v.at[p])
            for j in range(_BLK):
                pltpu.async_copy(
                    tab_hbm.at[idx_v.at[p, j]],
                    rows_v.at[p, pl.ds(j * _BL, _BL)], gsems[p])

        def wait_gathers(p):
            for j in range(_BLK):
                pltpu.make_async_copy(
                    tab_hbm.at[idx_v.at[p, j]],
                    rows_v.at[p, pl.ds(j * _BL, _BL)], gsems[p]).wait()

        def fire_writes(u, p):
            for su in range(_BLK):
                r = w_row0 + u * _BLK + su
                d1 = r % _D1
                bb = (r // _D1) % _BB
                t = r // (_D1 * _BB)
                d3 = t % _D3
                d2 = t // _D3
                for h in range(2):
                    pltpu.async_copy(
                        tiles_v.at[p, su, :, :, pl.ds(0, _BL)],
                        out_hbm.at[d1, d2, d3, pl.ds(4 * h, 4), bb],
                        wsems[p])

        def drain_writes(p):
            for su in range(_BLK):
                for h in range(2):
                    pltpu.make_async_copy(
                        tiles_v.at[p, su, :, :, pl.ds(0, _BL)],
                        out_hbm.at[0, 0, 0, pl.ds(4 * h, 4), 0],
                        wsems[p]).wait()

        ehp_lo, el_lo = lane // 8, lane % 8
        ehp_hi = ehp_lo + 2

        def transpose(p):
            for su in range(_BLK):
                def do_row(b, c):
                    for s in range(4):
                        bq = b * 4 + s
                        row = su * _BL + bq
                        bl_ids = jnp.full((16,), bq, jnp.int32)
                        v0 = rows_v[p, row, pl.ds(0, 16)] * _SCALE
                        v1 = rows_v[p, row, pl.ds(16, 16)] * _SCALE
                        plsc.store_scatter(tiles_v.at[p, su],
                                           [ehp_lo, el_lo, bl_ids], v0)
                        plsc.store_scatter(tiles_v.at[p, su],
                                           [ehp_hi, el_lo, bl_ids], v1)
                    return c
                lax.fori_loop(0, _BL // 4, do_row, 0)

        load_and_fire(0, 0)
        load_and_fire(1, 1)

        def do_pair(g, carry):
            for p in range(2):
                u = 2 * g + p
                wait_gathers(p)

                @pl.when(u >= 2)
                def _():
                    drain_writes(p)

                transpose(p)
                fire_writes(u, p)

                @pl.when(u + 2 < n_blocks)
                def _():
                    load_and_fire(u + 2, p)
            return carry

        lax.fori_loop(0, n_blocks // 2, do_pair, 0)
        drain_writes(0)
        drain_writes(1)

    return k(xp, table)


def kernel(x, embedding):
    xp = (x.reshape(_BB, _BL, _D1, _D2, _D3)
            .transpose(3, 4, 0, 2, 1)
            .reshape(_NROWS, _BL)
            .astype(jnp.int32))
    tail16 = embedding[_NFULL * _BL:, :].reshape(16, _BL)
    tbl = _sc_table_transpose(embedding.T, tail16).reshape(1000000, 32)
    out7 = _sc_impl(xp, tbl)
    out = (out7.transpose(4, 6, 0, 1, 2, 3, 5)
               .reshape(_BB * _BL, _D1, _D2, _D3, 64))
    return out


# speedup vs baseline: 2.4795x; 1.0993x over previous
import functools
import math

import jax
import jax.numpy as jnp
from jax import lax
from jax.experimental import pallas as pl
from jax.experimental.pallas import tpu as pltpu
from jax.experimental.pallas import tpu_sc as plsc

_SCALE = math.sqrt(32.0)
_D1, _D2, _D3 = 4, 10, 20
_BB, _BL = 8, 128
_NROWS = _D2 * _D3 * _BB * _D1
_BLK = 4

_E = 1_000_000
_NFULL = _E // 128
_REM = _E - _NFULL * 128


@jax.jit
def _sc_table_transpose(tab_t, tail16):
    info = plsc.get_sparse_core_info()
    num_workers = info.num_cores * info.num_subcores
    n_k = (_NFULL + num_workers - 1) // num_workers
    n_pairs = (n_k + 1) // 2

    mesh = plsc.VectorSubcoreMesh(core_axis_name="c", subcore_axis_name="s")

    @functools.partial(
        pl.kernel,
        out_type=jax.ShapeDtypeStruct((_E * 32,), jnp.float32),
        mesh=mesh,
        scratch_types=[
            pltpu.VMEM((2, 32, _BL), jnp.float32),
            pltpu.VMEM((32 * _BL,), jnp.float32),
            pltpu.VMEM((32 * _BL,), jnp.float32),
            pltpu.VMEM((_REM * 32,), jnp.float32),
            pltpu.SemaphoreType.DMA,
            pltpu.SemaphoreType.DMA,
            pltpu.SemaphoreType.DMA,
            pltpu.SemaphoreType.DMA,
        ],
        compiler_params=pltpu.CompilerParams(use_tc_tiling_on_sc=True,
                                             needs_layout_passes=False),
    )
    def tk(tab_hbm, tail_hbm, out_hbm, in_v, out_v0, out_v1, tail_v,
           isem0, isem1, wsem0, wsem1):
        out_vs = (out_v0, out_v1)
        isems = (isem0, isem1)
        wsems = (wsem0, wsem1)
        wid = lax.axis_index("s") * info.num_cores + lax.axis_index("c")
        lane = lax.broadcasted_iota(jnp.int32, (16,), 0)

        def group_of(k):
            return k * num_workers + wid

        def in_src(g):
            c0 = pl.multiple_of(g * _BL, _BL)
            return tab_hbm.at[:, pl.ds(c0, _BL)]

        def load(k, p):
            g = group_of(k)

            @pl.when(g < _NFULL)
            def _():
                pltpu.async_copy(in_src(g), in_v.at[p], isems[p])

        def wait_in(g, p):
            pltpu.make_async_copy(in_src(g), in_v.at[p], isems[p]).wait()

        def transpose(src, dst, ncols):
            for e0 in range(0, 32, 16):
                e_ids = e0 + lane
                fb = [((lane + d) & 15) * 32 + e_ids for d in range(16)]

                def do_blk(t, c):
                    f0 = t * 512

                    for d in range(16):
                        f = f0 + fb[d]
                        v = plsc.load_gather(src, [e_ids, f >> 5])
                        plsc.store_scatter(dst, [f], v)
                    return c
                lax.fori_loop(0, ncols // 16, do_blk, 0)

        def fire_write(g, p):
            r0 = pl.multiple_of(g * 4096, 4096)
            pltpu.async_copy(out_vs[p],
                             out_hbm.at[pl.ds(r0, 4096)], wsems[p])

        def drain_write(p):
            pltpu.make_async_copy(out_vs[p],
                                  out_hbm.at[pl.ds(0, 4096)], wsems[p]).wait()

        load(0, 0)
        load(1, 1)

        def do_pair(j, carry):
            for p in range(2):
                k = 2 * j + p
                g = group_of(k)

                @pl.when(g < _NFULL)
                def _():
                    wait_in(g, p)

                    @pl.when(k >= 2)
                    def _():
                        drain_write(p)

                    transpose(in_v.at[p], out_vs[p], _BL)
                    fire_write(g, p)
                load(k + 2, p)
            return carry

        lax.fori_loop(0, n_pairs, do_pair, 0)

        drain_write(0)
        drain_write(1)

        @pl.when(wid == 0)
        def _():
            pltpu.sync_copy(tail_hbm, tail_v)
            pltpu.sync_copy(tail_v,
                            out_hbm.at[pl.ds(_NFULL * _BL * 32, _REM * 32)])

    return tk(tab_t, tail16)


@jax.jit
def _sc_impl(xp, table):
    info = plsc.get_sparse_core_info()
    num_cores, num_subcores = info.num_cores, info.num_subcores
    num_workers = num_cores * num_subcores
    rows_per_w = _NROWS // num_workers
    n_blocks = rows_per_w // _BLK

    mesh = plsc.VectorSubcoreMesh(core_axis_name="c", subcore_axis_name="s")

    @functools.partial(
        pl.kernel,
        out_type=jax.ShapeDtypeStruct((_D1, _D2, _D3, 8, _BB, 8, _BL),
                                      jnp.float32),
        mesh=mesh,
        scratch_types=[
            pltpu.VMEM((2, _BLK, _BL), jnp.int32),
            pltpu.VMEM((2, _BLK * _BL, 32), jnp.float32),
            pltpu.VMEM((2, _BLK, 4, 8, 129), jnp.float32),
            pltpu.SemaphoreType.DMA,
            pltpu.SemaphoreType.DMA,
            pltpu.SemaphoreType.DMA,
            pltpu.SemaphoreType.DMA,
        ],
        compiler_params=pltpu.CompilerParams(use_tc_tiling_on_sc=False,
                                             needs_layout_passes=False),
    )
    def k(xp_hbm, tab_hbm, out_hbm, idx_v, rows_v, tiles_v,
          gsem0, gsem1, wsem0, wsem1):
        gsems = (gsem0, gsem1)
        wsems = (wsem0, wsem1)
        wid = lax.axis_index("s") * num_cores + lax.axis_index("c")
        w_row0 = wid * rows_per_w
        lane = lax.broadcasted_iota(jnp.int32, (16,), 0)

        def load_and_fire(u, p):
            r0 = w_row0 + u * _BLK
            pltpu.sync_copy(xp_hbm.at[pl.ds(r0, _BLK)], idx_v.at[p])
            for j in range(_BLK):
                pltpu.async_copy(
                    tab_hbm.at[idx_v.at[p, j]],
                    rows_v.at[p, pl.ds(j * _BL, _BL)], gsems[p])

        def wait_gathers(p):
            for j in range(_BLK):
                pltpu.make_async_copy(
                    tab_hbm.at[idx_v.at[p, j]],
                    rows_v.at[p, pl.ds(j * _BL, _BL)], gsems[p]).wait()

        def fire_writes(u, p):
            for su in range(_BLK):
                r = w_row0 + u * _BLK + su
                d1 = r % _D1
                bb = (r // _D1) % _BB
                t = r // (_D1 * _BB)
                d3 = t % _D3
                d2 = t // _D3
                for h in range(2):
                    pltpu.async_copy(
                        tiles_v.at[p, su, :, :, pl.ds(0, _BL)],
                        out_hbm.at[d1, d2, d3, pl.ds(4 * h, 4), bb],
                        wsems[p])

        def drain_writes(p):
            for su in range(_BLK):
                for h in range(2):
                    pltpu.make_async_copy(
                        tiles_v.at[p, su, :, :, pl.ds(0, _BL)],
                        out_hbm.at[0, 0, 0, pl.ds(4 * h, 4), 0],
                        wsems[p]).wait()

        ehp_lo, el_lo = lane // 8, lane % 8
        ehp_hi = ehp_lo + 2

        def transpose(p):
            for su in range(_BLK):
                def do_row(b, c):
                    for s in range(4):
                        bq = b * 4 + s
                        row = su * _BL + bq
                        bl_ids = jnp.full((16,), bq, jnp.int32)
                        v0 = rows_v[p, row, pl.ds(0, 16)] * _SCALE
                        v1 = rows_v[p, row, pl.ds(16, 16)] * _SCALE
                        plsc.store_scatter(tiles_v.at[p, su],
                                           [ehp_lo, el_lo, bl_ids], v0)
                        plsc.store_scatter(tiles_v.at[p, su],
                                           [ehp_hi, el_lo, bl_ids], v1)
                    return c
                lax.fori_loop(0, _BL // 4, do_row, 0)

        load_and_fire(0, 0)
        load_and_fire(1, 1)

        def do_pair(g, carry):
            for p in range(2):
                u = 2 * g + p
                wait_gathers(p)

                @pl.when(u >= 2)
                def _():
                    drain_writes(p)

                transpose(p)
                fire_writes(u, p)

                @pl.when(u + 2 < n_blocks)
                def _():
                    load_and_fire(u + 2, p)
            return carry

        lax.fori_loop(0, n_blocks // 2, do_pair, 0)
        drain_writes(0)
        drain_writes(1)

    return k(xp, table)


def kernel(x, embedding):
    xp = (x.reshape(_BB, _BL, _D1, _D2, _D3)
            .transpose(3, 4, 0, 2, 1)
            .reshape(_NROWS, _BL)
            .astype(jnp.int32))
    tail16 = embedding[_NFULL * _BL:, :].reshape(_REM * 32)
    tbl = _sc_table_transpose(embedding.T, tail16).reshape(1000000, 32)
    out7 = _sc_impl(xp, tbl)
    out = (out7.transpose(4, 6, 0, 1, 2, 3, 5)
               .reshape(_BB * _BL, _D1, _D2, _D3, 64))
    return out
